# initial kernel scaffold (unmeasured)
import jax
import jax.numpy as jnp
from jax import lax
from jax.experimental import pallas as pl
from jax.experimental.pallas import tpu as pltpu

N_DEV = 8
M = 8192
N = 4096
CH = M // N_DEV


def kernel(x, w_mat):
    m, k_sh = x.shape
    assert (m, k_sh) == (M, M // N_DEV)
    assert w_mat.shape == (M // N_DEV, N)

    def body(x_ref, w_ref, out_ref, send_ref, recv_ref, stage_ref,
             send_sems, recv_sems, copy_sems, credit_sem):
        d = lax.axis_index("i")
        left = lax.rem(d - 1 + N_DEV, N_DEV)
        right = lax.rem(d + 1, N_DEV)

        barrier = pltpu.get_barrier_semaphore()
        for nbr in (left, right):
            pl.semaphore_signal(barrier, inc=1, device_id=(nbr,),
                                device_id_type=pl.DeviceIdType.MESH)
        pl.semaphore_wait(barrier, 2)

        def partial(c):
            xs = x_ref[pl.ds(c * CH, CH), :]
            return jnp.dot(xs, w_ref[...], preferred_element_type=jnp.float32)

        pending = [None, None]
        n_writes = [0]

        def write_out(c, val_f32):
            slot = n_writes[0] % 2
            if pending[slot] is not None:
                pending[slot].wait()
            stage_ref[slot] = val_f32
            cp = pltpu.make_async_copy(
                stage_ref.at[slot],
                out_ref.at[pl.ds(c * CH, CH), :],
                copy_sems.at[slot],
            )
            cp.start()
            pending[slot] = cp
            n_writes[0] += 1

        send_ref[0] = partial(d).astype(jnp.bfloat16)

        for s in range(2 * (N_DEV - 1)):
            slot = s % 2
            if s >= 2:
                pl.semaphore_wait(credit_sem, 1)
            rdma = pltpu.make_async_remote_copy(
                src_ref=send_ref.at[slot],
                dst_ref=recv_ref.at[slot],
                send_sem=send_sems.at[slot],
                recv_sem=recv_sems.at[slot],
                device_id=(right,),
                device_id_type=pl.DeviceIdType.MESH,
            )
            rdma.start()
            rdma.wait()

            if s < N_DEV - 1:
                c = lax.rem(d - s - 1 + 2 * N_DEV, N_DEV)
                tot = recv_ref[slot].astype(jnp.float32) + partial(c)
                send_ref[(s + 1) % 2] = tot.astype(jnp.bfloat16)
                if s == N_DEV - 2:
                    write_out(c, tot)
            else:
                t = s - (N_DEV - 1)
                c = lax.rem(d - t + N_DEV, N_DEV)
                got = recv_ref[slot]
                if s < 2 * (N_DEV - 1) - 1:
                    send_ref[(s + 1) % 2] = got
                write_out(c, got.astype(jnp.float32))

            if s <= 2 * (N_DEV - 1) - 3:
                pl.semaphore_signal(credit_sem, inc=1, device_id=(left,),
                                    device_id_type=pl.DeviceIdType.MESH)

        for cp in pending:
            if cp is not None:
                cp.wait()

    return pl.pallas_call(
        body,
        out_shape=jax.ShapeDtypeStruct((M, N), jnp.float32),
        in_specs=[
            pl.BlockSpec(memory_space=pltpu.VMEM),
            pl.BlockSpec(memory_space=pltpu.VMEM),
        ],
        out_specs=pl.BlockSpec(memory_space=pltpu.ANY),
        scratch_shapes=[
            pltpu.VMEM((2, CH, N), jnp.bfloat16),
            pltpu.VMEM((2, CH, N), jnp.bfloat16),
            pltpu.VMEM((2, CH, N), jnp.float32),
            pltpu.SemaphoreType.DMA((2,)),
            pltpu.SemaphoreType.DMA((2,)),
            pltpu.SemaphoreType.DMA((2,)),
            pltpu.SemaphoreType.REGULAR,
        ],
        compiler_params=pltpu.CompilerParams(
            collective_id=0,
            vmem_limit_bytes=128 * 1024 * 1024,
        ),
    )(x, w_mat)


# baseline (device time: 1515154 ns/iter reference)
import jax
import jax.numpy as jnp
from jax import lax
from jax.experimental import pallas as pl
from jax.experimental.pallas import tpu as pltpu

N_DEV = 8
M = 8192
N = 4096
K_SH = M // N_DEV
CH = M // N_DEV
W_SL = 256
STRIP = 512
N_HOPS = 2 * (N_DEV - 1)


def kernel(x, w_mat):
    assert x.shape == (M, K_SH) and w_mat.shape == (K_SH, N)

    def body(x_hbm, w_hbm, out_ref, w_bf, w_stage, x_stage, xs_bf,
             acc_ref, send_ref, recv_ref,
             send_sem, recv_sem, x_sem, w_sem, out_sem, credit_sem):
        d = lax.axis_index("i")
        left = lax.rem(d - 1 + N_DEV, N_DEV)
        right = lax.rem(d + 1, N_DEV)

        barrier = pltpu.get_barrier_semaphore()
        for nbr in (left, right):
            pl.semaphore_signal(barrier, inc=1, device_id=(nbr,),
                                device_id_type=pl.DeviceIdType.MESH)
        pl.semaphore_wait(barrier, 2)

        for j in range(K_SH // W_SL):
            cp = pltpu.make_async_copy(
                w_hbm.at[pl.ds(j * W_SL, W_SL), :], w_stage, w_sem)
            cp.start()
            cp.wait()
            w_bf[pl.ds(j * W_SL, W_SL), :] = w_stage[...].astype(jnp.bfloat16)

        def load_x_chunk(c):
            cp = pltpu.make_async_copy(
                x_hbm.at[pl.ds(c * CH, CH), :], x_stage, x_sem)
            cp.start()
            cp.wait()
            xs_bf[...] = x_stage[...].astype(jnp.bfloat16)

        def accum_chunk(c, add_recv):
            load_x_chunk(c)
            for r in range(CH // STRIP):
                sl = pl.ds(r * STRIP, STRIP)
                p = jnp.dot(xs_bf[sl, :], w_bf[...],
                            preferred_element_type=jnp.float32)
                if add_recv:
                    p = p + recv_ref[sl, :].astype(jnp.float32)
                acc_ref[sl, :] = p

        def flush_acc(c):
            pltpu.make_async_copy(
                acc_ref, out_ref.at[pl.ds(c * CH, CH), :], out_sem).start()

        def wait_flush():
            pltpu.make_async_copy(
                acc_ref, out_ref.at[pl.ds(0, CH), :], out_sem).wait()

        accum_chunk(d, add_recv=False)
        send_ref[...] = acc_ref[...].astype(jnp.bfloat16)

        def signal_credit():
            pl.semaphore_signal(credit_sem, inc=1, device_id=(left,),
                                device_id_type=pl.DeviceIdType.MESH)

        def hop(s, carry):
            @pl.when(s >= 1)
            def _():
                pl.semaphore_wait(credit_sem, 1)

            rdma = pltpu.make_async_remote_copy(
                src_ref=send_ref, dst_ref=recv_ref,
                send_sem=send_sem, recv_sem=recv_sem,
                device_id=(right,), device_id_type=pl.DeviceIdType.MESH,
            )
            rdma.start()
            rdma.wait()

            @pl.when(s < N_DEV - 1)
            def _rs():
                c = lax.rem(d - s - 1 + 2 * N_DEV, N_DEV)
                accum_chunk(c, add_recv=True)
                signal_credit()
                send_ref[...] = acc_ref[...].astype(jnp.bfloat16)

                @pl.when(s == N_DEV - 2)
                def _():
                    flush_acc(c)

            @pl.when(s >= N_DEV - 1)
            def _ag():
                c = lax.rem(d - (s - (N_DEV - 1)) + N_DEV, N_DEV)

                @pl.when(s < N_HOPS - 1)
                def _():
                    send_ref[...] = recv_ref[...]

                wait_flush()
                acc_ref[...] = recv_ref[...].astype(jnp.float32)

                @pl.when(s < N_HOPS - 1)
                def _():
                    signal_credit()

                flush_acc(c)

            return carry

        lax.fori_loop(0, N_HOPS, hop, 0)
        wait_flush()

    return pl.pallas_call(
        body,
        out_shape=jax.ShapeDtypeStruct((M, N), jnp.float32),
        in_specs=[
            pl.BlockSpec(memory_space=pl.ANY),
            pl.BlockSpec(memory_space=pl.ANY),
        ],
        out_specs=pl.BlockSpec(memory_space=pl.ANY),
        scratch_shapes=[
            pltpu.VMEM((K_SH, N), jnp.bfloat16),
            pltpu.VMEM((W_SL, N), jnp.float32),
            pltpu.VMEM((CH, K_SH), jnp.float32),
            pltpu.VMEM((CH, K_SH), jnp.bfloat16),
            pltpu.VMEM((CH, N), jnp.float32),
            pltpu.VMEM((CH, N), jnp.bfloat16),
            pltpu.VMEM((CH, N), jnp.bfloat16),
            pltpu.SemaphoreType.DMA,
            pltpu.SemaphoreType.DMA,
            pltpu.SemaphoreType.DMA,
            pltpu.SemaphoreType.DMA,
            pltpu.SemaphoreType.DMA,
            pltpu.SemaphoreType.REGULAR,
        ],
        compiler_params=pltpu.CompilerParams(
            collective_id=0,
            vmem_limit_bytes=64 * 1024 * 1024,
        ),
    )(x, w_mat)


# device time: 819740 ns/iter; 1.8483x vs baseline; 1.8483x over previous
import jax
import jax.numpy as jnp
from jax import lax
from jax.experimental import pallas as pl
from jax.experimental.pallas import tpu as pltpu

N_DEV = 8
M = 8192
N = 4096
K_SH = M // N_DEV
CH = M // N_DEV
HALF = CH // 2
W_SL = 256
N_HOPS = 2 * (N_DEV - 1)


def kernel(x, w_mat):
    assert x.shape == (M, K_SH) and w_mat.shape == (K_SH, N)

    def body(x_hbm, w_hbm, out_ref, w_bf, w_stage, x_stage, xs_bf,
             acc_ref, sendR, sendL, recvR, recvL,
             sendR_sem, sendL_sem, recvR_sem, recvL_sem,
             x_sems, w_sem, outR_sem, outL_sem, creditR, creditL):
        d = lax.axis_index("i")
        left = lax.rem(d - 1 + N_DEV, N_DEV)
        right = lax.rem(d + 1, N_DEV)

        barrier = pltpu.get_barrier_semaphore()
        for nbr in (left, right):
            pl.semaphore_signal(barrier, inc=1, device_id=(nbr,),
                                device_id_type=pl.DeviceIdType.MESH)
        pl.semaphore_wait(barrier, 2)

        for j in range(K_SH // W_SL):
            cp = pltpu.make_async_copy(
                w_hbm.at[pl.ds(j * W_SL, W_SL), :], w_stage, w_sem)
            cp.start()
            cp.wait()
            w_bf[pl.ds(j * W_SL, W_SL), :] = w_stage[...].astype(jnp.bfloat16)

        def gemm_halves(cR, cL):
            cpR = pltpu.make_async_copy(
                x_hbm.at[pl.ds(cR * CH, HALF), :], x_stage.at[0],
                x_sems.at[0])
            cpL = pltpu.make_async_copy(
                x_hbm.at[pl.ds(cL * CH + HALF, HALF), :], x_stage.at[1],
                x_sems.at[1])
            cpR.start()
            cpL.start()
            cpR.wait()
            cpL.wait()
            xs_bf[0] = x_stage[0].astype(jnp.bfloat16)
            xs_bf[1] = x_stage[1].astype(jnp.bfloat16)
            acc_ref[pl.ds(0, HALF), :] = jnp.dot(
                xs_bf[0], w_bf[...], preferred_element_type=jnp.float32)
            acc_ref[pl.ds(HALF, HALF), :] = jnp.dot(
                xs_bf[1], w_bf[...], preferred_element_type=jnp.float32)

        def flush(cR, cL):
            pltpu.make_async_copy(
                acc_ref.at[pl.ds(0, HALF), :],
                out_ref.at[pl.ds(cR * CH, HALF), :], outR_sem).start()
            pltpu.make_async_copy(
                acc_ref.at[pl.ds(HALF, HALF), :],
                out_ref.at[pl.ds(cL * CH + HALF, HALF), :], outL_sem).start()

        def wait_flush():
            pltpu.make_async_copy(
                acc_ref.at[pl.ds(0, HALF), :],
                out_ref.at[pl.ds(0, HALF), :], outR_sem).wait()
            pltpu.make_async_copy(
                acc_ref.at[pl.ds(HALF, HALF), :],
                out_ref.at[pl.ds(HALF, HALF), :], outL_sem).wait()

        gemm_halves(d, d)
        sendR[...] = acc_ref[pl.ds(0, HALF), :].astype(jnp.bfloat16)
        sendL[...] = acc_ref[pl.ds(HALF, HALF), :].astype(jnp.bfloat16)

        def hop(s, carry):
            @pl.when(s >= 1)
            def _():
                pl.semaphore_wait(creditR, 1)
                pl.semaphore_wait(creditL, 1)

            rdmaR = pltpu.make_async_remote_copy(
                src_ref=sendR, dst_ref=recvR,
                send_sem=sendR_sem, recv_sem=recvR_sem,
                device_id=(right,), device_id_type=pl.DeviceIdType.MESH,
            )
            rdmaL = pltpu.make_async_remote_copy(
                src_ref=sendL, dst_ref=recvL,
                send_sem=sendL_sem, recv_sem=recvL_sem,
                device_id=(left,), device_id_type=pl.DeviceIdType.MESH,
            )
            rdmaR.start()
            rdmaL.start()

            @pl.when(s < N_DEV - 1)
            def _rs():
                cR = lax.rem(d - s - 1 + 2 * N_DEV, N_DEV)
                cL = lax.rem(d + s + 1, N_DEV)
                gemm_halves(cR, cL)
                rdmaR.wait()
                rdmaL.wait()
                totR = acc_ref[pl.ds(0, HALF), :] + recvR[...].astype(jnp.float32)
                totL = acc_ref[pl.ds(HALF, HALF), :] + recvL[...].astype(jnp.float32)
                acc_ref[pl.ds(0, HALF), :] = totR
                acc_ref[pl.ds(HALF, HALF), :] = totL
                sendR[...] = totR.astype(jnp.bfloat16)
                sendL[...] = totL.astype(jnp.bfloat16)
                pl.semaphore_signal(creditR, inc=1, device_id=(left,),
                                    device_id_type=pl.DeviceIdType.MESH)
                pl.semaphore_signal(creditL, inc=1, device_id=(right,),
                                    device_id_type=pl.DeviceIdType.MESH)

                @pl.when(s == N_DEV - 2)
                def _():
                    flush(cR, cL)

            @pl.when(s >= N_DEV - 1)
            def _ag():
                t = s - (N_DEV - 1)
                cR = lax.rem(d - t + N_DEV, N_DEV)
                cL = lax.rem(d + t, N_DEV)
                rdmaR.wait()
                rdmaL.wait()

                @pl.when(s < N_HOPS - 1)
                def _():
                    sendR[...] = recvR[...]
                    sendL[...] = recvL[...]

                wait_flush()
                acc_ref[pl.ds(0, HALF), :] = recvR[...].astype(jnp.float32)
                acc_ref[pl.ds(HALF, HALF), :] = recvL[...].astype(jnp.float32)

                @pl.when(s < N_HOPS - 1)
                def _():
                    pl.semaphore_signal(creditR, inc=1, device_id=(left,),
                                        device_id_type=pl.DeviceIdType.MESH)
                    pl.semaphore_signal(creditL, inc=1, device_id=(right,),
                                        device_id_type=pl.DeviceIdType.MESH)

                flush(cR, cL)

            return carry

        lax.fori_loop(0, N_HOPS, hop, 0)
        wait_flush()

    return pl.pallas_call(
        body,
        out_shape=jax.ShapeDtypeStruct((M, N), jnp.float32),
        in_specs=[
            pl.BlockSpec(memory_space=pl.ANY),
            pl.BlockSpec(memory_space=pl.ANY),
        ],
        out_specs=pl.BlockSpec(memory_space=pl.ANY),
        scratch_shapes=[
            pltpu.VMEM((K_SH, N), jnp.bfloat16),
            pltpu.VMEM((W_SL, N), jnp.float32),
            pltpu.VMEM((2, HALF, K_SH), jnp.float32),
            pltpu.VMEM((2, HALF, K_SH), jnp.bfloat16),
            pltpu.VMEM((CH, N), jnp.float32),
            pltpu.VMEM((HALF, N), jnp.bfloat16),
            pltpu.VMEM((HALF, N), jnp.bfloat16),
            pltpu.VMEM((HALF, N), jnp.bfloat16),
            pltpu.VMEM((HALF, N), jnp.bfloat16),
            pltpu.SemaphoreType.DMA,
            pltpu.SemaphoreType.DMA,
            pltpu.SemaphoreType.DMA,
            pltpu.SemaphoreType.DMA,
            pltpu.SemaphoreType.DMA((2,)),
            pltpu.SemaphoreType.DMA,
            pltpu.SemaphoreType.DMA,
            pltpu.SemaphoreType.DMA,
            pltpu.SemaphoreType.REGULAR,
            pltpu.SemaphoreType.REGULAR,
        ],
        compiler_params=pltpu.CompilerParams(
            collective_id=0,
            vmem_limit_bytes=64 * 1024 * 1024,
        ),
    )(x, w_mat)


# device time: 819013 ns/iter; 1.8500x vs baseline; 1.0009x over previous
import jax
import jax.numpy as jnp
from jax import lax
from jax.experimental import pallas as pl
from jax.experimental.pallas import tpu as pltpu

N_DEV = 8
M = 8192
N = 4096
K_SH = M // N_DEV
CH = M // N_DEV
HALF = CH // 2
W_SL = 256
N_HOPS = 2 * (N_DEV - 1)


def kernel(x, w_mat):
    assert x.shape == (M, K_SH) and w_mat.shape == (K_SH, N)

    def body(x_hbm, w_hbm, out_ref, w_bf, w_stage, x_stage, xs_bf,
             acc_ref, sendR, sendL, recvR, recvL,
             sendR_sem, sendL_sem, recvR_sem, recvL_sem,
             x_sems, w_sem, outR_sem, outL_sem, creditR, creditL):
        d = lax.axis_index("i")
        left = lax.rem(d - 1 + N_DEV, N_DEV)
        right = lax.rem(d + 1, N_DEV)

        barrier = pltpu.get_barrier_semaphore()
        for nbr in (left, right):
            pl.semaphore_signal(barrier, inc=1, device_id=(nbr,),
                                device_id_type=pl.DeviceIdType.MESH)
        pl.semaphore_wait(barrier, 2)

        def start_x_loads(cR, cL):
            cpR = pltpu.make_async_copy(
                x_hbm.at[pl.ds(cR * CH, HALF), :], x_stage.at[0],
                x_sems.at[0])
            cpL = pltpu.make_async_copy(
                x_hbm.at[pl.ds(cL * CH + HALF, HALF), :], x_stage.at[1],
                x_sems.at[1])
            cpR.start()
            cpL.start()
            return cpR, cpL

        def gemm_halves(cR, cL, loads=None):
            cpR, cpL = loads if loads is not None else start_x_loads(cR, cL)
            cpR.wait()
            cpL.wait()
            xs_bf[0] = x_stage[0].astype(jnp.bfloat16)
            xs_bf[1] = x_stage[1].astype(jnp.bfloat16)
            acc_ref[pl.ds(0, HALF), :] = jnp.dot(
                xs_bf[0], w_bf[...], preferred_element_type=jnp.float32)
            acc_ref[pl.ds(HALF, HALF), :] = jnp.dot(
                xs_bf[1], w_bf[...], preferred_element_type=jnp.float32)

        def flush(cR, cL):
            pltpu.make_async_copy(
                acc_ref.at[pl.ds(0, HALF), :],
                out_ref.at[pl.ds(cR * CH, HALF), :], outR_sem).start()
            pltpu.make_async_copy(
                acc_ref.at[pl.ds(HALF, HALF), :],
                out_ref.at[pl.ds(cL * CH + HALF, HALF), :], outL_sem).start()

        def wait_flush():
            pltpu.make_async_copy(
                acc_ref.at[pl.ds(0, HALF), :],
                out_ref.at[pl.ds(0, HALF), :], outR_sem).wait()
            pltpu.make_async_copy(
                acc_ref.at[pl.ds(HALF, HALF), :],
                out_ref.at[pl.ds(HALF, HALF), :], outL_sem).wait()

        seed_loads = start_x_loads(d, d)

        for j in range(K_SH // W_SL):
            cp = pltpu.make_async_copy(
                w_hbm.at[pl.ds(j * W_SL, W_SL), :], w_stage, w_sem)
            cp.start()
            cp.wait()
            w_bf[pl.ds(j * W_SL, W_SL), :] = w_stage[...].astype(jnp.bfloat16)

        gemm_halves(d, d, seed_loads)
        sendR[...] = acc_ref[pl.ds(0, HALF), :].astype(jnp.bfloat16)
        sendL[...] = acc_ref[pl.ds(HALF, HALF), :].astype(jnp.bfloat16)

        def hop(s, carry):
            @pl.when(s >= 1)
            def _():
                pl.semaphore_wait(creditR, 1)
                pl.semaphore_wait(creditL, 1)

            rdmaR = pltpu.make_async_remote_copy(
                src_ref=sendR, dst_ref=recvR,
                send_sem=sendR_sem, recv_sem=recvR_sem,
                device_id=(right,), device_id_type=pl.DeviceIdType.MESH,
            )
            rdmaL = pltpu.make_async_remote_copy(
                src_ref=sendL, dst_ref=recvL,
                send_sem=sendL_sem, recv_sem=recvL_sem,
                device_id=(left,), device_id_type=pl.DeviceIdType.MESH,
            )
            rdmaR.start()
            rdmaL.start()

            @pl.when(s < N_DEV - 1)
            def _rs():
                cR = lax.rem(d - s - 1 + 2 * N_DEV, N_DEV)
                cL = lax.rem(d + s + 1, N_DEV)
                gemm_halves(cR, cL)
                rdmaR.wait()
                rdmaL.wait()
                totR = acc_ref[pl.ds(0, HALF), :] + recvR[...].astype(jnp.float32)
                totL = acc_ref[pl.ds(HALF, HALF), :] + recvL[...].astype(jnp.float32)
                acc_ref[pl.ds(0, HALF), :] = totR
                acc_ref[pl.ds(HALF, HALF), :] = totL
                sendR[...] = totR.astype(jnp.bfloat16)
                sendL[...] = totL.astype(jnp.bfloat16)
                pl.semaphore_signal(creditR, inc=1, device_id=(left,),
                                    device_id_type=pl.DeviceIdType.MESH)
                pl.semaphore_signal(creditL, inc=1, device_id=(right,),
                                    device_id_type=pl.DeviceIdType.MESH)

                @pl.when(s == N_DEV - 2)
                def _():
                    flush(cR, cL)

            @pl.when(s >= N_DEV - 1)
            def _ag():
                t = s - (N_DEV - 1)
                cR = lax.rem(d - t + N_DEV, N_DEV)
                cL = lax.rem(d + t, N_DEV)
                rdmaR.wait()
                rdmaL.wait()

                @pl.when(s < N_HOPS - 1)
                def _():
                    sendR[...] = recvR[...]
                    sendL[...] = recvL[...]

                wait_flush()
                acc_ref[pl.ds(0, HALF), :] = recvR[...].astype(jnp.float32)
                acc_ref[pl.ds(HALF, HALF), :] = recvL[...].astype(jnp.float32)

                @pl.when(s < N_HOPS - 1)
                def _():
                    pl.semaphore_signal(creditR, inc=1, device_id=(left,),
                                        device_id_type=pl.DeviceIdType.MESH)
                    pl.semaphore_signal(creditL, inc=1, device_id=(right,),
                                        device_id_type=pl.DeviceIdType.MESH)

                flush(cR, cL)

            return carry

        lax.fori_loop(0, N_HOPS, hop, 0)
        wait_flush()

    return pl.pallas_call(
        body,
        out_shape=jax.ShapeDtypeStruct((M, N), jnp.float32),
        in_specs=[
            pl.BlockSpec(memory_space=pl.ANY),
            pl.BlockSpec(memory_space=pl.ANY),
        ],
        out_specs=pl.BlockSpec(memory_space=pl.ANY),
        scratch_shapes=[
            pltpu.VMEM((K_SH, N), jnp.bfloat16),
            pltpu.VMEM((W_SL, N), jnp.float32),
            pltpu.VMEM((2, HALF, K_SH), jnp.float32),
            pltpu.VMEM((2, HALF, K_SH), jnp.bfloat16),
            pltpu.VMEM((CH, N), jnp.float32),
            pltpu.VMEM((HALF, N), jnp.bfloat16),
            pltpu.VMEM((HALF, N), jnp.bfloat16),
            pltpu.VMEM((HALF, N), jnp.bfloat16),
            pltpu.VMEM((HALF, N), jnp.bfloat16),
            pltpu.SemaphoreType.DMA,
            pltpu.SemaphoreType.DMA,
            pltpu.SemaphoreType.DMA,
            pltpu.SemaphoreType.DMA,
            pltpu.SemaphoreType.DMA((2,)),
            pltpu.SemaphoreType.DMA,
            pltpu.SemaphoreType.DMA,
            pltpu.SemaphoreType.DMA,
            pltpu.SemaphoreType.REGULAR,
            pltpu.SemaphoreType.REGULAR,
        ],
        compiler_params=pltpu.CompilerParams(
            collective_id=0,
            vmem_limit_bytes=64 * 1024 * 1024,
        ),
    )(x, w_mat)


# device time: 815710 ns/iter; 1.8575x vs baseline; 1.0040x over previous
import jax
import jax.numpy as jnp
from jax import lax
from jax.experimental import pallas as pl
from jax.experimental.pallas import tpu as pltpu

N_DEV = 8
M = 8192
N = 4096
K_SH = M // N_DEV
CH = M // N_DEV
HALF = CH // 2
W_SL = 256
N_HOPS = 2 * (N_DEV - 1)


def kernel(x, w_mat):
    assert x.shape == (M, K_SH) and w_mat.shape == (K_SH, N)

    def body(x_hbm, w_hbm, out_ref, w_bf, w_stage, x_stage, xs_bf,
             acc_ref, sendR, sendL, recvR, recvL,
             sendR_sem, sendL_sem, recvR_sem, recvL_sem,
             x_sems, w_sem, outR_sem, outL_sem, creditR, creditL):
        d = lax.axis_index("i")
        left = lax.rem(d - 1 + N_DEV, N_DEV)
        right = lax.rem(d + 1, N_DEV)

        barrier = pltpu.get_barrier_semaphore()
        for nbr in (left, right):
            pl.semaphore_signal(barrier, inc=1, device_id=(nbr,),
                                device_id_type=pl.DeviceIdType.MESH)
        pl.semaphore_wait(barrier, 2)

        def start_x_loads(cR, cL):
            cpR = pltpu.make_async_copy(
                x_hbm.at[pl.ds(cR * CH, HALF), :], x_stage.at[0],
                x_sems.at[0])
            cpL = pltpu.make_async_copy(
                x_hbm.at[pl.ds(cL * CH + HALF, HALF), :], x_stage.at[1],
                x_sems.at[1])
            cpR.start()
            cpL.start()
            return cpR, cpL

        def gemm_halves(cR, cL, loads=None):
            cpR, cpL = loads if loads is not None else start_x_loads(cR, cL)
            cpR.wait()
            cpL.wait()
            xs_bf[0] = x_stage[0].astype(jnp.bfloat16)
            xs_bf[1] = x_stage[1].astype(jnp.bfloat16)
            acc_ref[pl.ds(0, HALF), :] = jnp.dot(
                xs_bf[0], w_bf[...], preferred_element_type=jnp.float32)
            acc_ref[pl.ds(HALF, HALF), :] = jnp.dot(
                xs_bf[1], w_bf[...], preferred_element_type=jnp.float32)

        def flush(cR, cL):
            pltpu.make_async_copy(
                acc_ref.at[pl.ds(0, HALF), :],
                out_ref.at[pl.ds(cR * CH, HALF), :], outR_sem).start()
            pltpu.make_async_copy(
                acc_ref.at[pl.ds(HALF, HALF), :],
                out_ref.at[pl.ds(cL * CH + HALF, HALF), :], outL_sem).start()

        def wait_flush():
            pltpu.make_async_copy(
                acc_ref.at[pl.ds(0, HALF), :],
                out_ref.at[pl.ds(0, HALF), :], outR_sem).wait()
            pltpu.make_async_copy(
                acc_ref.at[pl.ds(HALF, HALF), :],
                out_ref.at[pl.ds(HALF, HALF), :], outL_sem).wait()

        seed_loads = start_x_loads(d, d)

        for j in range(K_SH // W_SL):
            cp = pltpu.make_async_copy(
                w_hbm.at[pl.ds(j * W_SL, W_SL), :], w_stage, w_sem)
            cp.start()
            cp.wait()
            w_bf[pl.ds(j * W_SL, W_SL), :] = w_stage[...].astype(jnp.bfloat16)

        gemm_halves(d, d, seed_loads)
        sendR[...] = acc_ref[pl.ds(0, HALF), :].astype(jnp.bfloat16)
        sendL[...] = acc_ref[pl.ds(HALF, HALF), :].astype(jnp.bfloat16)

        def hop(s, carry):
            @pl.when(s >= 1)
            def _():
                pl.semaphore_wait(creditR, 1)
                pl.semaphore_wait(creditL, 1)

            def make_rdmas(srcR, dstR, srcL, dstL):
                rR = pltpu.make_async_remote_copy(
                    src_ref=srcR, dst_ref=dstR,
                    send_sem=sendR_sem, recv_sem=recvR_sem,
                    device_id=(right,), device_id_type=pl.DeviceIdType.MESH,
                )
                rL = pltpu.make_async_remote_copy(
                    src_ref=srcL, dst_ref=dstL,
                    send_sem=sendL_sem, recv_sem=recvL_sem,
                    device_id=(left,), device_id_type=pl.DeviceIdType.MESH,
                )
                return rR, rL

            ag_t = s - (N_DEV - 1)
            fwd_swapped = jnp.logical_and(s >= N_DEV, lax.rem(ag_t, 2) == 1)

            @pl.when(jnp.logical_not(fwd_swapped))
            def _():
                rR, rL = make_rdmas(sendR, recvR, sendL, recvL)
                rR.start()
                rL.start()

            @pl.when(fwd_swapped)
            def _():
                rR, rL = make_rdmas(recvR, sendR, recvL, sendL)
                rR.start()
                rL.start()

            rdmaR, rdmaL = make_rdmas(sendR, recvR, sendL, recvL)

            @pl.when(s < N_DEV - 1)
            def _rs():
                cR = lax.rem(d - s - 1 + 2 * N_DEV, N_DEV)
                cL = lax.rem(d + s + 1, N_DEV)
                gemm_halves(cR, cL)
                rdmaR.wait()
                rdmaL.wait()
                totR = acc_ref[pl.ds(0, HALF), :] + recvR[...].astype(jnp.float32)
                totL = acc_ref[pl.ds(HALF, HALF), :] + recvL[...].astype(jnp.float32)
                acc_ref[pl.ds(0, HALF), :] = totR
                acc_ref[pl.ds(HALF, HALF), :] = totL
                sendR[...] = totR.astype(jnp.bfloat16)
                sendL[...] = totL.astype(jnp.bfloat16)
                pl.semaphore_signal(creditR, inc=1, device_id=(left,),
                                    device_id_type=pl.DeviceIdType.MESH)
                pl.semaphore_signal(creditL, inc=1, device_id=(right,),
                                    device_id_type=pl.DeviceIdType.MESH)

                @pl.when(s == N_DEV - 2)
                def _():
                    flush(cR, cL)

            @pl.when(s >= N_DEV - 1)
            def _ag():
                t = s - (N_DEV - 1)
                cR = lax.rem(d - t + N_DEV, N_DEV)
                cL = lax.rem(d + t, N_DEV)
                rdmaR.wait()
                rdmaL.wait()
                wait_flush()

                @pl.when(lax.rem(t, 2) == 0)
                def _():
                    acc_ref[pl.ds(0, HALF), :] = recvR[...].astype(jnp.float32)
                    acc_ref[pl.ds(HALF, HALF), :] = recvL[...].astype(jnp.float32)

                @pl.when(lax.rem(t, 2) == 1)
                def _():
                    acc_ref[pl.ds(0, HALF), :] = sendR[...].astype(jnp.float32)
                    acc_ref[pl.ds(HALF, HALF), :] = sendL[...].astype(jnp.float32)

                @pl.when(s < N_HOPS - 1)
                def _():
                    pl.semaphore_signal(creditR, inc=1, device_id=(left,),
                                        device_id_type=pl.DeviceIdType.MESH)
                    pl.semaphore_signal(creditL, inc=1, device_id=(right,),
                                        device_id_type=pl.DeviceIdType.MESH)

                flush(cR, cL)

            return carry

        lax.fori_loop(0, N_HOPS, hop, 0)
        wait_flush()

    return pl.pallas_call(
        body,
        out_shape=jax.ShapeDtypeStruct((M, N), jnp.float32),
        in_specs=[
            pl.BlockSpec(memory_space=pl.ANY),
            pl.BlockSpec(memory_space=pl.ANY),
        ],
        out_specs=pl.BlockSpec(memory_space=pl.ANY),
        scratch_shapes=[
            pltpu.VMEM((K_SH, N), jnp.bfloat16),
            pltpu.VMEM((W_SL, N), jnp.float32),
            pltpu.VMEM((2, HALF, K_SH), jnp.float32),
            pltpu.VMEM((2, HALF, K_SH), jnp.bfloat16),
            pltpu.VMEM((CH, N), jnp.float32),
            pltpu.VMEM((HALF, N), jnp.bfloat16),
            pltpu.VMEM((HALF, N), jnp.bfloat16),
            pltpu.VMEM((HALF, N), jnp.bfloat16),
            pltpu.VMEM((HALF, N), jnp.bfloat16),
            pltpu.SemaphoreType.DMA,
            pltpu.SemaphoreType.DMA,
            pltpu.SemaphoreType.DMA,
            pltpu.SemaphoreType.DMA,
            pltpu.SemaphoreType.DMA((2,)),
            pltpu.SemaphoreType.DMA,
            pltpu.SemaphoreType.DMA,
            pltpu.SemaphoreType.DMA,
            pltpu.SemaphoreType.REGULAR,
            pltpu.SemaphoreType.REGULAR,
        ],
        compiler_params=pltpu.CompilerParams(
            collective_id=0,
            vmem_limit_bytes=64 * 1024 * 1024,
        ),
    )(x, w_mat)
